# baseline (speedup 1.0000x reference)
import jax
import jax.numpy as jnp
from jax.experimental import pallas as pl
from jax.experimental.pallas import tpu as pltpu

_EPSILON = 0.9


def _select_kernel(w_ref, mask_ref, rand_ref, out_ref):
    w = w_ref[...]
    kk = w.shape[1]
    m = jnp.max(w, axis=1, keepdims=True)
    ii = jax.lax.broadcasted_iota(jnp.int32, w.shape, 1)
    amax = jnp.min(jnp.where(w == m, ii, kk), axis=1, keepdims=True)
    out_ref[...] = jnp.where(mask_ref[...] != 0, rand_ref[...], amax)


def _reparam_kernel(idx_ref, mu_ref, lv_ref, eps_ref,
                    samp_ref, cmu_ref, clv_ref):
    mu = mu_ref[0]
    lv = lv_ref[0]
    std = jnp.exp(lv * 0.5)
    cmu_ref[0] = mu
    clv_ref[0] = lv
    samp_ref[0] = mu + std * eps_ref[0]


def kernel(epoch, means, log_vars, weights):
    b, k, d = means.shape
    rkey = jax.random.key(42)
    kmask, kidx, keps = jax.random.split(rkey, 3)
    mask = (jax.random.uniform(kmask, (b,)) < _EPSILON).astype(jnp.int32)
    rand_idx = jax.random.randint(kidx, (b,), 0, k)
    eps = jax.random.normal(keps, (b, d), dtype=jnp.float32)

    chosen = pl.pallas_call(
        _select_kernel,
        out_shape=jax.ShapeDtypeStruct((b, 1), jnp.int32),
    )(weights, mask[:, None], rand_idx[:, None])
    chosen_indices = chosen[:, 0]

    means_r = means.reshape(b * k, 1, d)
    lv_r = log_vars.reshape(b * k, 1, d)
    eps_r = eps.reshape(b, 1, d)
    grid_spec = pltpu.PrefetchScalarGridSpec(
        num_scalar_prefetch=1,
        grid=(b,),
        in_specs=[
            pl.BlockSpec((1, 1, d), lambda i, idx: (i * k + idx[i], 0, 0)),
            pl.BlockSpec((1, 1, d), lambda i, idx: (i * k + idx[i], 0, 0)),
            pl.BlockSpec((1, 1, d), lambda i, idx: (i, 0, 0)),
        ],
        out_specs=[
            pl.BlockSpec((1, 1, d), lambda i, idx: (i, 0, 0)),
            pl.BlockSpec((1, 1, d), lambda i, idx: (i, 0, 0)),
            pl.BlockSpec((1, 1, d), lambda i, idx: (i, 0, 0)),
        ],
    )
    sample, chosen_mu, chosen_lv = pl.pallas_call(
        _reparam_kernel,
        grid_spec=grid_spec,
        out_shape=[
            jax.ShapeDtypeStruct((b, 1, d), jnp.float32),
            jax.ShapeDtypeStruct((b, 1, d), jnp.float32),
            jax.ShapeDtypeStruct((b, 1, d), jnp.float32),
        ],
    )(chosen_indices, means_r, lv_r, eps_r)

    return (sample[:, 0, :], chosen_indices, chosen_mu[:, 0, :], chosen_lv[:, 0, :])


# speedup vs baseline: 22.8837x; 22.8837x over previous
import functools

import jax
import jax.numpy as jnp
from jax import lax
from jax.experimental import pallas as pl
from jax.experimental.pallas import tpu as pltpu
from jax.experimental.pallas import tpu_sc as plsc

_EPSILON = 0.9
_NC = 2
_NS = 16
_L = 16


def _sc_select_gather(b, k, d):
    nw = _NC * _NS
    bw = b // nw
    ng = bw // _L

    mesh = plsc.VectorSubcoreMesh(core_axis_name="c", subcore_axis_name="s")

    @functools.partial(
        pl.kernel,
        mesh=mesh,
        out_type=[
            jax.ShapeDtypeStruct((b,), jnp.int32),
            jax.ShapeDtypeStruct((b, d), jnp.float32),
            jax.ShapeDtypeStruct((b, d), jnp.float32),
        ],
        scratch_types=[
            pltpu.VMEM((k, bw), jnp.float32),
            pltpu.VMEM((bw,), jnp.int32),
            pltpu.VMEM((bw,), jnp.int32),
            pltpu.VMEM((_L,), jnp.int32),
            pltpu.VMEM((_L, d), jnp.float32),
            pltpu.VMEM((_L, d), jnp.float32),
            pltpu.SemaphoreType.DMA,
            pltpu.SemaphoreType.DMA,
        ],
    )
    def sc_kernel(wt_hbm, sel_hbm, means_hbm, lv_hbm,
                  cidx_hbm, cmu_hbm, clv_hbm,
                  wt_v, sel_v, cho_v, row_v, mu_v, lvv_v, sem_a, sem_b):
        wid = lax.axis_index("s") * _NC + lax.axis_index("c")
        base = wid * bw
        pltpu.sync_copy(wt_hbm.at[wid], wt_v)
        pltpu.sync_copy(sel_hbm.at[wid], sel_v)
        for t in range(ng):
            sl = pl.ds(t * _L, _L)
            best = wt_v[0, sl]
            besti = jnp.zeros((_L,), jnp.int32)
            for h in range(1, k):
                v = wt_v[h, sl]
                upd = v > best
                besti = jnp.where(upd, h, besti)
                best = jnp.where(upd, v, best)
            sel = sel_v[sl]
            chosen = jnp.where(sel >= 0, sel, besti)
            cho_v[sl] = chosen
            tok = base + t * _L + lax.iota(jnp.int32, _L)
            row_v[...] = tok * k + chosen
            ga = pltpu.async_copy(means_hbm.at[row_v], mu_v, sem_a)
            gb = pltpu.async_copy(lv_hbm.at[row_v], lvv_v, sem_b)
            ga.wait()
            gb.wait()
            pltpu.sync_copy(mu_v, cmu_hbm.at[pl.ds(base + t * _L, _L)])
            pltpu.sync_copy(lvv_v, clv_hbm.at[pl.ds(base + t * _L, _L)])
        pltpu.sync_copy(cho_v, cidx_hbm.at[pl.ds(base, bw)])

    return sc_kernel


def _reparam_body(mu_ref, lv_ref, eps_ref, samp_ref):
    lv = lv_ref[...]
    samp_ref[...] = mu_ref[...] + jnp.exp(lv * 0.5) * eps_ref[...]


def kernel(epoch, means, log_vars, weights):
    b, k, d = means.shape
    nw = _NC * _NS
    bw = b // nw

    rkey = jax.random.key(42)
    kmask, kidx, keps = jax.random.split(rkey, 3)
    mask = jax.random.uniform(kmask, (b,)) < _EPSILON
    rand_idx = jax.random.randint(kidx, (b,), 0, k)
    eps = jax.random.normal(keps, (b, d), dtype=jnp.float32)

    wt3 = weights.T.reshape(k, nw, bw).transpose(1, 0, 2)
    sel3 = jnp.where(mask, rand_idx, -1).astype(jnp.int32).reshape(nw, bw)
    means2 = means.reshape(b * k, d)
    lv2 = log_vars.reshape(b * k, d)

    sc = _sc_select_gather(b, k, d)
    chosen_indices, chosen_mu, chosen_lv = sc(wt3, sel3, means2, lv2)

    rb = 256
    sample = pl.pallas_call(
        _reparam_body,
        grid=(b // rb,),
        in_specs=[
            pl.BlockSpec((rb, d), lambda i: (i, 0)),
            pl.BlockSpec((rb, d), lambda i: (i, 0)),
            pl.BlockSpec((rb, d), lambda i: (i, 0)),
        ],
        out_specs=pl.BlockSpec((rb, d), lambda i: (i, 0)),
        out_shape=jax.ShapeDtypeStruct((b, d), jnp.float32),
    )(chosen_mu, chosen_lv, eps)

    return (sample, chosen_indices, chosen_mu, chosen_lv)
